# baseline (speedup 1.0000x reference)
import functools

import jax
import jax.numpy as jnp
from jax.experimental import pallas as pl
from jax.experimental.pallas import tpu as pltpu

Q_SIZE = 1024
K_SIZE = 100000
D_SIZE = 64
K_BLK = 2000


def _split3(v):
    u = jax.lax.bitcast_convert_type(v, jnp.uint32)
    hi = jax.lax.bitcast_convert_type(u & jnp.uint32(0xFFFF0000), jnp.float32)
    r1 = v - hi
    u1 = jax.lax.bitcast_convert_type(r1, jnp.uint32)
    mid = jax.lax.bitcast_convert_type(u1 & jnp.uint32(0xFFFF0000), jnp.float32)
    r2 = r1 - mid
    return (hi.astype(jnp.bfloat16), mid.astype(jnp.bfloat16),
            r2.astype(jnp.bfloat16))


def _nn_kernel(xt_ref, mus_ref, out_ref, runmin_ref, runblk_ref, *, nblk):
    pid = pl.program_id(0)

    mus_blk = mus_ref[...]
    x_a, x_b, x_c = _split3(xt_ref[...])
    x_aug = jnp.concatenate([x_a, x_a, x_b, x_c, x_a, x_b], axis=0)
    m_a, m_b, m_c = _split3(mus_blk)
    m_aug = jnp.concatenate([m_a, m_b, m_a, m_a, m_c, m_b], axis=1)
    st = jax.lax.dot_general(
        m_aug, x_aug, (((1,), (0,)), ((), ())),
        preferred_element_type=jnp.float32,
    )
    norms = jnp.sum(mus_blk * mus_blk, axis=1, keepdims=True)
    scores = norms + st

    @pl.when(pid == 0)
    def _init():
        runmin_ref[...] = scores
        runblk_ref[...] = jnp.zeros((K_BLK, Q_SIZE), jnp.int32)

    @pl.when(pid > 0)
    def _merge():
        old = runmin_ref[...]
        better = scores < old
        runblk_ref[...] = jnp.where(better, pid, runblk_ref[...])
        runmin_ref[...] = jnp.where(better, scores, old)

    @pl.when(pid == nblk - 1)
    def _emit():
        rm = runmin_ref[...]
        m = jnp.min(rm, axis=0, keepdims=True)
        row = jax.lax.broadcasted_iota(jnp.int32, (K_BLK, Q_SIZE), 0)
        gk = runblk_ref[...] * K_BLK + row
        gk = jnp.where(rm == m, gk, jnp.int32(0x7FFFFFFF))
        out_ref[...] = jnp.min(gk, axis=0, keepdims=True)


def kernel(X, mus):
    nblk = K_SIZE // K_BLK
    xt = -2.0 * X.T

    out = pl.pallas_call(
        functools.partial(_nn_kernel, nblk=nblk),
        grid=(nblk,),
        in_specs=[
            pl.BlockSpec((D_SIZE, Q_SIZE), lambda i: (0, 0)),
            pl.BlockSpec((K_BLK, D_SIZE), lambda i: (i, 0)),
        ],
        out_specs=pl.BlockSpec((1, Q_SIZE), lambda i: (0, 0)),
        out_shape=jax.ShapeDtypeStruct((1, Q_SIZE), jnp.int32),
        scratch_shapes=[
            pltpu.VMEM((K_BLK, Q_SIZE), jnp.float32),
            pltpu.VMEM((K_BLK, Q_SIZE), jnp.int32),
        ],
    )(xt, mus)
    return out.reshape(Q_SIZE)


# speedup vs baseline: 1.5315x; 1.5315x over previous
import functools

import jax
import jax.numpy as jnp
from jax.experimental import pallas as pl
from jax.experimental.pallas import tpu as pltpu

Q_SIZE = 1024
K_SIZE = 100000
D_SIZE = 64
K_BLK = 4000


def _split3(v):
    u = jax.lax.bitcast_convert_type(v, jnp.uint32)
    hi = jax.lax.bitcast_convert_type(u & jnp.uint32(0xFFFF0000), jnp.float32)
    r1 = v - hi
    u1 = jax.lax.bitcast_convert_type(r1, jnp.uint32)
    mid = jax.lax.bitcast_convert_type(u1 & jnp.uint32(0xFFFF0000), jnp.float32)
    r2 = r1 - mid
    return (hi.astype(jnp.bfloat16), mid.astype(jnp.bfloat16),
            r2.astype(jnp.bfloat16))


def _nn_kernel(xt_ref, mus_ref, out_ref, min_ref, arg_ref, xaug_ref, *, nblk):
    pid = pl.program_id(0)

    @pl.when(pid == 0)
    def _prep():
        x_a, x_b, x_c = _split3(xt_ref[...])
        xaug_ref[...] = jnp.concatenate([x_a, x_a, x_b, x_c, x_a, x_b],
                                        axis=0)

    mus_blk = mus_ref[...]
    m_a, m_b, m_c = _split3(mus_blk)
    m_aug = jnp.concatenate([m_a, m_b, m_a, m_a, m_c, m_b], axis=1)
    st = jax.lax.dot_general(
        m_aug, xaug_ref[...], (((1,), (0,)), ((), ())),
        preferred_element_type=jnp.float32,
    )
    norms = jnp.sum(mus_blk * mus_blk, axis=1, keepdims=True)
    scores = norms + st

    blk_min = jnp.min(scores, axis=0, keepdims=True)
    blk_arg = jnp.argmin(scores, axis=0).astype(jnp.int32)[None, :]

    @pl.when(pid == 0)
    def _init():
        min_ref[...] = jnp.full((1, Q_SIZE), jnp.inf, jnp.float32)
        arg_ref[...] = jnp.zeros((1, Q_SIZE), jnp.int32)

    better = blk_min < min_ref[...]
    arg_ref[...] = jnp.where(better, blk_arg + pid * K_BLK, arg_ref[...])
    min_ref[...] = jnp.where(better, blk_min, min_ref[...])

    @pl.when(pid == nblk - 1)
    def _emit():
        out_ref[...] = arg_ref[...]


def kernel(X, mus):
    nblk = K_SIZE // K_BLK
    xt = -2.0 * X.T

    out = pl.pallas_call(
        functools.partial(_nn_kernel, nblk=nblk),
        grid=(nblk,),
        in_specs=[
            pl.BlockSpec((D_SIZE, Q_SIZE), lambda i: (0, 0)),
            pl.BlockSpec((K_BLK, D_SIZE), lambda i: (i, 0)),
        ],
        out_specs=pl.BlockSpec((1, Q_SIZE), lambda i: (0, 0)),
        out_shape=jax.ShapeDtypeStruct((1, Q_SIZE), jnp.int32),
        scratch_shapes=[
            pltpu.VMEM((1, Q_SIZE), jnp.float32),
            pltpu.VMEM((1, Q_SIZE), jnp.int32),
            pltpu.VMEM((6 * D_SIZE, Q_SIZE), jnp.bfloat16),
        ],
    )(xt, mus)
    return out.reshape(Q_SIZE)


# speedup vs baseline: 1.5440x; 1.0081x over previous
import functools

import jax
import jax.numpy as jnp
from jax.experimental import pallas as pl
from jax.experimental.pallas import tpu as pltpu

Q_SIZE = 1024
K_SIZE = 100000
D_SIZE = 64
K_BLK = 4000


def _split3(v):
    u = jax.lax.bitcast_convert_type(v, jnp.uint32)
    hi = jax.lax.bitcast_convert_type(u & jnp.uint32(0xFFFF0000), jnp.float32)
    r1 = v - hi
    u1 = jax.lax.bitcast_convert_type(r1, jnp.uint32)
    mid = jax.lax.bitcast_convert_type(u1 & jnp.uint32(0xFFFF0000), jnp.float32)
    r2 = r1 - mid
    return (hi.astype(jnp.bfloat16), mid.astype(jnp.bfloat16),
            r2.astype(jnp.bfloat16))


def _nn_kernel(xt_ref, mus_ref, out_ref, min_ref, arg_ref, xaug_ref, *, nblk):
    pid = pl.program_id(0)

    @pl.when(pid == 0)
    def _prep():
        x_a, x_b, x_c = _split3(xt_ref[...])
        ones = jnp.ones((3, Q_SIZE), jnp.bfloat16)
        xaug_ref[...] = jnp.concatenate([x_a, x_a, x_b, x_c, x_a, x_b, ones],
                                        axis=0)

    mus_blk = mus_ref[...]
    m_a, m_b, m_c = _split3(mus_blk)
    norms = jnp.sum(mus_blk * mus_blk, axis=1, keepdims=True)
    n_a, n_b, n_c = _split3(norms)
    m_aug = jnp.concatenate([m_a, m_b, m_a, m_a, m_c, m_b, n_a, n_b, n_c],
                            axis=1)
    scores = jax.lax.dot_general(
        m_aug, xaug_ref[...], (((1,), (0,)), ((), ())),
        preferred_element_type=jnp.float32,
    )

    blk_min = jnp.min(scores, axis=0, keepdims=True)
    blk_arg = jnp.argmin(scores, axis=0).astype(jnp.int32)[None, :]

    @pl.when(pid == 0)
    def _init():
        min_ref[...] = jnp.full((1, Q_SIZE), jnp.inf, jnp.float32)
        arg_ref[...] = jnp.zeros((1, Q_SIZE), jnp.int32)

    better = blk_min < min_ref[...]
    arg_ref[...] = jnp.where(better, blk_arg + pid * K_BLK, arg_ref[...])
    min_ref[...] = jnp.where(better, blk_min, min_ref[...])

    @pl.when(pid == nblk - 1)
    def _emit():
        out_ref[...] = arg_ref[...]


def kernel(X, mus):
    nblk = K_SIZE // K_BLK
    xt = -2.0 * X.T

    out = pl.pallas_call(
        functools.partial(_nn_kernel, nblk=nblk),
        grid=(nblk,),
        in_specs=[
            pl.BlockSpec((D_SIZE, Q_SIZE), lambda i: (0, 0)),
            pl.BlockSpec((K_BLK, D_SIZE), lambda i: (i, 0)),
        ],
        out_specs=pl.BlockSpec((1, Q_SIZE), lambda i: (0, 0)),
        out_shape=jax.ShapeDtypeStruct((1, Q_SIZE), jnp.int32),
        scratch_shapes=[
            pltpu.VMEM((1, Q_SIZE), jnp.float32),
            pltpu.VMEM((1, Q_SIZE), jnp.int32),
            pltpu.VMEM((6 * D_SIZE + 3, Q_SIZE), jnp.bfloat16),
        ],
    )(xt, mus)
    return out.reshape(Q_SIZE)
